# Initial kernel scaffold; baseline (speedup 1.0000x reference)
import jax
import jax.numpy as jnp
from jax.experimental import pallas as pl


def kernel(x, pos_table):
    raise NotImplementedError("write your pallas kernel here")


# speedup vs baseline: 2.1835x; 2.1835x over previous
import jax
import jax.numpy as jnp
from jax.experimental import pallas as pl

_BS = 256


def _add_pe_kernel(x_ref, pe_ref, o_ref):
    o_ref[...] = x_ref[...] + pe_ref[...][:, None, :]


def kernel(x, pos_table):
    S, B, D = x.shape
    pe = pos_table[:S]
    return pl.pallas_call(
        _add_pe_kernel,
        grid=(S // _BS,),
        in_specs=[
            pl.BlockSpec((_BS, B, D), lambda i: (i, 0, 0)),
            pl.BlockSpec((_BS, D), lambda i: (i, 0)),
        ],
        out_specs=pl.BlockSpec((_BS, B, D), lambda i: (i, 0, 0)),
        out_shape=jax.ShapeDtypeStruct((S, B, D), x.dtype),
    )(x, pe)
